# Initial kernel scaffold; baseline (speedup 1.0000x reference)
import jax
import jax.numpy as jnp
from jax.experimental import pallas as pl


def kernel(x, edge_index, edge_weight, W, b):
    raise NotImplementedError("write your pallas kernel here")


# speedup vs baseline: 2.9174x; 2.9174x over previous
import functools

import jax
import jax.numpy as jnp
from jax import lax
from jax.experimental import pallas as pl
from jax.experimental.pallas import tpu as pltpu
from jax.experimental.pallas import tpu_sc as plsc

NC = 2
NS = 16
CH = 80


def _linear_split(x, W, b, blk=1000):
    n, d = x.shape
    dh = d // NC

    def body(x_ref, w_ref, b_ref, o_ref):
        o_ref[...] = lax.dot_general(
            x_ref[...], w_ref[...], (((1,), (1,)), ((), ())),
            preferred_element_type=jnp.float32) + b_ref[0]

    nb = n // blk
    return pl.pallas_call(
        body,
        grid=(NC, nb),
        in_specs=[
            pl.BlockSpec((blk, d), lambda p, i: (i, 0)),
            pl.BlockSpec((dh, d), lambda p, i: (p, 0)),
            pl.BlockSpec((1, 1, dh), lambda p, i: (p, 0, 0)),
        ],
        out_specs=pl.BlockSpec((blk, dh), lambda p, i: (p * nb + i, 0)),
        out_shape=jax.ShapeDtypeStruct((NC * n, dh), jnp.float32),
    )(x, W, b.reshape(NC, 1, dh))


def _build_scatter(n, e, dh):
    ept = e // NS
    nchunk = ept // CH
    rpt = n // NS
    zr = 125
    nz = rpt // zr

    @functools.partial(
        pl.kernel,
        out_type=jax.ShapeDtypeStruct((NC * NS, n // NS, dh), jnp.float32),
        mesh=plsc.VectorSubcoreMesh(core_axis_name="c", subcore_axis_name="s"),
        scratch_types=[
            pltpu.VMEM((ept,), jnp.int32),
            pltpu.VMEM((nchunk, CH), jnp.int32),
            pltpu.VMEM((ept,), jnp.float32),
            pltpu.VMEM((CH, dh), jnp.float32),
            pltpu.VMEM((zr, dh), jnp.float32),
            pltpu.VMEM_SHARED((n, dh), jnp.float32),
            pltpu.SemaphoreType.DMA,
        ],
        compiler_params=pltpu.CompilerParams(use_tc_tiling_on_sc=False),
    )
    def sc_scatter(h2, col_h, row_h, w_h, out_h, col_v, row_v, w_v, buf, zb,
                   acc, sem):
        c = lax.axis_index("c")
        s = lax.axis_index("s")
        zero16 = jnp.zeros((16,), jnp.float32)

        def zrow(i, carry):
            for q in range(dh // 16):
                zb[i, pl.ds(q * 16, 16)] = zero16
            return carry
        lax.fori_loop(0, zr, zrow, None)
        for k in range(nz):
            pltpu.sync_copy(zb, acc.at[pl.ds(s * rpt + k * zr, zr)])

        base = s * ept
        pltpu.sync_copy(col_h.at[pl.ds(base, ept)], col_v)
        pltpu.sync_copy(row_h.at[s], row_v)
        pltpu.sync_copy(w_h.at[pl.ds(base, ept)], w_v)

        shift = jnp.full((16,), c * n, jnp.int32)

        def sh(i, carry):
            col_v[pl.ds(i * 16, 16)] = col_v[pl.ds(i * 16, 16)] + shift
            return carry
        lax.fori_loop(0, ept // 16, sh, None)

        plsc.subcore_barrier()

        def chunk(j, carry):
            pltpu.async_copy(h2.at[col_v.at[pl.ds(j * CH, CH)]], buf,
                             sem).wait()

            def sgroup(g, carry2):
                rbase = g * 16
                wg = w_v[pl.ds(j * CH + rbase, 16)]
                for t in range(16):
                    wv = lax.gather(
                        wg, jnp.full((16, 1), t, jnp.int32),
                        lax.GatherDimensionNumbers(
                            offset_dims=(), collapsed_slice_dims=(0,),
                            start_index_map=(0,)),
                        slice_sizes=(1,),
                        mode=lax.GatherScatterMode.PROMISE_IN_BOUNDS)
                    for q in range(dh // 16):
                        buf[rbase + t, pl.ds(q * 16, 16)] = (
                            buf[rbase + t, pl.ds(q * 16, 16)] * wv)
                return carry2
            lax.fori_loop(0, CH // 16, sgroup, None)

            pltpu.sync_copy(buf, acc.at[row_v.at[j]], add=True)
            return carry
        lax.fori_loop(0, nchunk, chunk, None)

        plsc.subcore_barrier()
        pltpu.sync_copy(acc.at[pl.ds(s * rpt, rpt)], out_h.at[c * NS + s])

    return sc_scatter


def kernel(x, edge_index, edge_weight, W, b):
    n, d = x.shape
    e = edge_weight.shape[0]
    dh = d // NC
    col = edge_index[1].astype(jnp.int32)
    row2 = edge_index[0].astype(jnp.int32).reshape(NS, e // (NS * CH), CH)
    h2 = _linear_split(x, W, b)
    out2 = _build_scatter(n, e, dh)(h2, col, row2, edge_weight)
    out2 = out2.reshape(NC, n, dh)
    return jnp.concatenate([out2[0], out2[1]], axis=1)


# speedup vs baseline: 4.0687x; 1.3946x over previous
import functools

import jax
import jax.numpy as jnp
from jax import lax
from jax.experimental import pallas as pl
from jax.experimental.pallas import tpu as pltpu
from jax.experimental.pallas import tpu_sc as plsc

NC = 2
NS = 16
CH = 80


def _linear_split(x, W, b, blk=1000):
    n, d = x.shape
    dh = d // NC

    def body(x_ref, w_ref, b_ref, o_ref):
        o_ref[...] = lax.dot_general(
            x_ref[...], w_ref[...], (((1,), (1,)), ((), ())),
            preferred_element_type=jnp.float32) + b_ref[0]

    nb = n // blk
    return pl.pallas_call(
        body,
        grid=(NC, nb),
        in_specs=[
            pl.BlockSpec((blk, d), lambda p, i: (i, 0)),
            pl.BlockSpec((dh, d), lambda p, i: (p, 0)),
            pl.BlockSpec((1, 1, dh), lambda p, i: (p, 0, 0)),
        ],
        out_specs=pl.BlockSpec((blk, dh), lambda p, i: (p * nb + i, 0)),
        out_shape=jax.ShapeDtypeStruct((NC * n, dh), jnp.float32),
    )(x, W, b.reshape(NC, 1, dh))


def _build_scatter(n, e, dh):
    ept = e // NS
    nchunk = ept // CH
    rpt = n // NS
    zr = 125
    nz = rpt // zr

    @functools.partial(
        pl.kernel,
        out_type=jax.ShapeDtypeStruct((NC * NS, n // NS, dh), jnp.float32),
        mesh=plsc.VectorSubcoreMesh(core_axis_name="c", subcore_axis_name="s"),
        scratch_types=[
            pltpu.VMEM((ept,), jnp.int32),
            pltpu.VMEM((nchunk, CH), jnp.int32),
            pltpu.VMEM((ept,), jnp.float32),
            pltpu.VMEM((2, CH, dh), jnp.float32),
            pltpu.VMEM((zr, dh), jnp.float32),
            pltpu.VMEM_SHARED((n, dh), jnp.float32),
            pltpu.SemaphoreType.DMA,
            pltpu.SemaphoreType.DMA,
            pltpu.SemaphoreType.DMA,
            pltpu.SemaphoreType.DMA,
        ],
        compiler_params=pltpu.CompilerParams(use_tc_tiling_on_sc=False),
    )
    def sc_scatter(h2, col_h, row_h, w_h, out_h, col_v, row_v, w_v, buf2, zb,
                   acc, gsem0, gsem1, ssem0, ssem1):
        gsem = (gsem0, gsem1)
        ssem = (ssem0, ssem1)
        c = lax.axis_index("c")
        s = lax.axis_index("s")
        zero16 = jnp.zeros((16,), jnp.float32)

        def zrow(i, carry):
            for q in range(dh // 16):
                zb[i, pl.ds(q * 16, 16)] = zero16
            return carry
        lax.fori_loop(0, zr, zrow, None)
        for k in range(nz):
            pltpu.sync_copy(zb, acc.at[pl.ds(s * rpt + k * zr, zr)])

        base = s * ept
        pltpu.sync_copy(col_h.at[pl.ds(base, ept)], col_v)
        pltpu.sync_copy(row_h.at[s], row_v)
        pltpu.sync_copy(w_h.at[pl.ds(base, ept)], w_v)

        shift = jnp.full((16,), c * n, jnp.int32)

        def sh(i, carry):
            col_v[pl.ds(i * 16, 16)] = col_v[pl.ds(i * 16, 16)] + shift
            return carry
        lax.fori_loop(0, ept // 16, sh, None)

        plsc.subcore_barrier()

        def start_gather(j, p):
            pltpu.async_copy(h2.at[col_v.at[pl.ds(j * CH, CH)]], buf2.at[p],
                             gsem[p])

        def wait_gather(j, p):
            pltpu.make_async_copy(h2.at[col_v.at[pl.ds(j * CH, CH)]],
                                  buf2.at[p], gsem[p]).wait()

        def start_scatter(j, p):
            pltpu.async_copy(buf2.at[p], acc.at[row_v.at[j]], ssem[p],
                             add=True)

        def wait_scatter(j, p):
            pltpu.make_async_copy(buf2.at[p], acc.at[row_v.at[j]],
                                  ssem[p]).wait()

        def scale(j, p):
            def sgroup(g, carry2):
                rbase = g * 16
                wg = w_v[pl.ds(j * CH + rbase, 16)]
                for t in range(16):
                    wv = lax.gather(
                        wg, jnp.full((16, 1), t, jnp.int32),
                        lax.GatherDimensionNumbers(
                            offset_dims=(), collapsed_slice_dims=(0,),
                            start_index_map=(0,)),
                        slice_sizes=(1,),
                        mode=lax.GatherScatterMode.PROMISE_IN_BOUNDS)
                    for q in range(dh // 16):
                        buf2[p, rbase + t, pl.ds(q * 16, 16)] = (
                            buf2[p, rbase + t, pl.ds(q * 16, 16)] * wv)
                return carry2
            lax.fori_loop(0, CH // 16, sgroup, None)

        def step(j, p, first=False, last=False):
            if not first:
                wait_scatter(j - 1, 1 - p)
            if not last:
                start_gather(j + 1, 1 - p)
            wait_gather(j, p)
            scale(j, p)
            start_scatter(j, p)

        start_gather(jnp.int32(0), 0)
        step(jnp.int32(0), 0, first=True)

        def pair(jj, carry):
            step(2 * jj + 1, 1)
            step(2 * jj + 2, 0)
            return carry
        lax.fori_loop(0, (nchunk - 2) // 2, pair, None)

        step(jnp.int32(nchunk - 1), 1, last=True)
        wait_scatter(jnp.int32(nchunk - 1), 1)

        plsc.subcore_barrier()
        pltpu.sync_copy(acc.at[pl.ds(s * rpt, rpt)], out_h.at[c * NS + s])

    return sc_scatter


def kernel(x, edge_index, edge_weight, W, b):
    n, d = x.shape
    e = edge_weight.shape[0]
    dh = d // NC
    col = edge_index[1].astype(jnp.int32)
    row2 = edge_index[0].astype(jnp.int32).reshape(NS, e // (NS * CH), CH)
    h2 = _linear_split(x, W, b)
    out2 = _build_scatter(n, e, dh)(h2, col, row2, edge_weight)
    out2 = out2.reshape(NC, n, dh)
    return jnp.concatenate([out2[0], out2[1]], axis=1)


# speedup vs baseline: 7.2871x; 1.7910x over previous
import functools

import jax
import jax.numpy as jnp
from jax import lax
from jax.experimental import pallas as pl
from jax.experimental.pallas import tpu as pltpu
from jax.experimental.pallas import tpu_sc as plsc

NC = 2
NS = 16
CH = 80


def _linear_split(x, W, b, blk=1000):
    n, d = x.shape
    dh = d // NC

    def body(x_ref, w_ref, b_ref, o_ref):
        o_ref[...] = lax.dot_general(
            x_ref[...], w_ref[...], (((1,), (1,)), ((), ())),
            preferred_element_type=jnp.float32) + b_ref[0]

    nb = n // blk
    return pl.pallas_call(
        body,
        grid=(NC, nb),
        in_specs=[
            pl.BlockSpec((blk, d), lambda p, i: (i, 0)),
            pl.BlockSpec((dh, d), lambda p, i: (p, 0)),
            pl.BlockSpec((1, 1, dh), lambda p, i: (p, 0, 0)),
        ],
        out_specs=pl.BlockSpec((blk, dh), lambda p, i: (p * nb + i, 0)),
        out_shape=jax.ShapeDtypeStruct((NC * n, dh), jnp.float32),
    )(x, W, b.reshape(NC, 1, dh))


def _build_scatter(n, e, dh):
    ept = e // NS
    nchunk = ept // CH
    rpt = n // NS
    zr = 125
    nz = rpt // zr

    @functools.partial(
        pl.kernel,
        out_type=jax.ShapeDtypeStruct((NC * NS, n // NS, dh), jnp.float32),
        mesh=plsc.VectorSubcoreMesh(core_axis_name="c", subcore_axis_name="s"),
        scratch_types=[
            pltpu.VMEM((ept,), jnp.int32),
            pltpu.VMEM((nchunk, CH), jnp.int32),
            pltpu.VMEM((ept,), jnp.float32),
            pltpu.VMEM((2, CH, dh), jnp.float32),
            pltpu.VMEM((zr, dh), jnp.float32),
            pltpu.VMEM_SHARED((n, dh), jnp.float32),
            pltpu.SemaphoreType.DMA,
            pltpu.SemaphoreType.DMA,
            pltpu.SemaphoreType.DMA,
            pltpu.SemaphoreType.DMA,
        ],
        compiler_params=pltpu.CompilerParams(use_tc_tiling_on_sc=False),
    )
    def sc_scatter(h2, col_h, row_h, w_h, out_h, col_v, row_v, w_v, buf2, zb,
                   acc, gsem0, gsem1, ssem0, ssem1):
        gsem = (gsem0, gsem1)
        ssem = (ssem0, ssem1)
        c = lax.axis_index("c")
        s = lax.axis_index("s")
        zero16 = jnp.zeros((16,), jnp.float32)

        def zrow(i, carry):
            for q in range(dh // 16):
                zb[i, pl.ds(q * 16, 16)] = zero16
            return carry
        lax.fori_loop(0, zr, zrow, None)
        for k in range(nz):
            pltpu.sync_copy(zb, acc.at[pl.ds(s * rpt + k * zr, zr)])

        base = s * ept
        pltpu.sync_copy(col_h.at[pl.ds(base, ept)], col_v)
        pltpu.sync_copy(row_h.at[s], row_v)
        pltpu.sync_copy(w_h.at[pl.ds(base, ept)], w_v)

        shift = jnp.full((16,), c * n, jnp.int32)

        def sh(i, carry):
            col_v[pl.ds(i * 16, 16)] = col_v[pl.ds(i * 16, 16)] + shift
            return carry
        lax.fori_loop(0, ept // 16, sh, None)

        plsc.subcore_barrier()

        def start_gather(j, p):
            pltpu.async_copy(h2.at[col_v.at[pl.ds(j * CH, CH)]], buf2.at[p],
                             gsem[p])

        def wait_gather(j, p):
            pltpu.make_async_copy(h2.at[col_v.at[pl.ds(j * CH, CH)]],
                                  buf2.at[p], gsem[p]).wait()

        def start_scatter(j, p):
            pltpu.async_copy(buf2.at[p], acc.at[row_v.at[j]], ssem[p],
                             add=True)

        def wait_scatter(j, p):
            pltpu.make_async_copy(buf2.at[p], acc.at[row_v.at[j]],
                                  ssem[p]).wait()

        def scale(j, p):
            for g in range(CH // 16):
                wg = w_v[pl.ds(j * CH + g * 16, 16)]
                for t in range(16):
                    wv = lax.gather(
                        wg, jnp.full((16, 1), t, jnp.int32),
                        lax.GatherDimensionNumbers(
                            offset_dims=(), collapsed_slice_dims=(0,),
                            start_index_map=(0,)),
                        slice_sizes=(1,),
                        mode=lax.GatherScatterMode.PROMISE_IN_BOUNDS)
                    r = g * 16 + t
                    for q in range(dh // 16):
                        buf2[p, r, pl.ds(q * 16, 16)] = (
                            buf2[p, r, pl.ds(q * 16, 16)] * wv)

        def step(j, p, first=False, last=False):
            if not first:
                wait_scatter(j - 1, 1 - p)
            if not last:
                start_gather(j + 1, 1 - p)
            wait_gather(j, p)
            scale(j, p)
            start_scatter(j, p)

        start_gather(jnp.int32(0), 0)
        step(jnp.int32(0), 0, first=True)

        def pair(jj, carry):
            step(2 * jj + 1, 1)
            step(2 * jj + 2, 0)
            return carry
        lax.fori_loop(0, (nchunk - 2) // 2, pair, None)

        step(jnp.int32(nchunk - 1), 1, last=True)
        wait_scatter(jnp.int32(nchunk - 1), 1)

        plsc.subcore_barrier()
        pltpu.sync_copy(acc.at[pl.ds(s * rpt, rpt)], out_h.at[c * NS + s])

    return sc_scatter


def kernel(x, edge_index, edge_weight, W, b):
    n, d = x.shape
    e = edge_weight.shape[0]
    dh = d // NC
    col = edge_index[1].astype(jnp.int32)
    row2 = edge_index[0].astype(jnp.int32).reshape(NS, e // (NS * CH), CH)
    h2 = _linear_split(x, W, b)
    out2 = _build_scatter(n, e, dh)(h2, col, row2, edge_weight)
    out2 = out2.reshape(NC, n, dh)
    return jnp.concatenate([out2[0], out2[1]], axis=1)


# speedup vs baseline: 8.7155x; 1.1960x over previous
import functools

import jax
import jax.numpy as jnp
from jax import lax
from jax.experimental import pallas as pl
from jax.experimental.pallas import tpu as pltpu
from jax.experimental.pallas import tpu_sc as plsc

NC = 2
NS = 16
CH = 80


def _linear_split(x, W, b, blk=1000):
    n, d = x.shape
    dh = d // NC

    def body(x_ref, w_ref, b_ref, o_ref):
        o_ref[...] = lax.dot_general(
            x_ref[...], w_ref[...], (((1,), (1,)), ((), ())),
            preferred_element_type=jnp.float32) + b_ref[0]

    nb = n // blk
    return pl.pallas_call(
        body,
        grid=(NC, nb),
        in_specs=[
            pl.BlockSpec((blk, d), lambda p, i: (i, 0)),
            pl.BlockSpec((dh, d), lambda p, i: (p, 0)),
            pl.BlockSpec((1, 1, dh), lambda p, i: (p, 0, 0)),
        ],
        out_specs=pl.BlockSpec((blk, dh), lambda p, i: (p * nb + i, 0)),
        out_shape=jax.ShapeDtypeStruct((NC * n, dh), jnp.float32),
    )(x, W, b.reshape(NC, 1, dh))


def _build_scatter(n, e, dh):
    ept = e // NS
    nchunk = ept // CH
    rpt = n // NS
    zr = 125
    nz = rpt // zr

    @functools.partial(
        pl.kernel,
        out_type=jax.ShapeDtypeStruct((NC * NS, n // NS, dh), jnp.float32),
        mesh=plsc.VectorSubcoreMesh(core_axis_name="c", subcore_axis_name="s"),
        scratch_types=[
            pltpu.VMEM((ept,), jnp.int32),
            pltpu.VMEM((nchunk, CH), jnp.int32),
            pltpu.VMEM((ept,), jnp.float32),
            pltpu.VMEM((3, CH, dh), jnp.float32),
            pltpu.VMEM((zr, dh), jnp.float32),
            pltpu.VMEM_SHARED((n, dh), jnp.float32),
            pltpu.SemaphoreType.DMA,
            pltpu.SemaphoreType.DMA,
            pltpu.SemaphoreType.DMA,
            pltpu.SemaphoreType.DMA,
            pltpu.SemaphoreType.DMA,
            pltpu.SemaphoreType.DMA,
        ],
        compiler_params=pltpu.CompilerParams(use_tc_tiling_on_sc=False),
    )
    def sc_scatter(h2, col_h, row_h, w_h, out_h, col_v, row_v, w_v, buf2, zb,
                   acc, gsem0, gsem1, gsem2, ssem0, ssem1, ssem2):
        gsem = (gsem0, gsem1, gsem2)
        ssem = (ssem0, ssem1, ssem2)
        c = lax.axis_index("c")
        s = lax.axis_index("s")
        zero16 = jnp.zeros((16,), jnp.float32)

        def zrow(i, carry):
            for q in range(dh // 16):
                zb[i, pl.ds(q * 16, 16)] = zero16
            return carry
        lax.fori_loop(0, zr, zrow, None)
        for k in range(nz):
            pltpu.sync_copy(zb, acc.at[pl.ds(s * rpt + k * zr, zr)])

        base = s * ept
        pltpu.sync_copy(col_h.at[pl.ds(base, ept)], col_v)
        pltpu.sync_copy(row_h.at[s], row_v)
        pltpu.sync_copy(w_h.at[pl.ds(base, ept)], w_v)

        shift = jnp.full((16,), c * n, jnp.int32)

        def sh(i, carry):
            col_v[pl.ds(i * 16, 16)] = col_v[pl.ds(i * 16, 16)] + shift
            return carry
        lax.fori_loop(0, ept // 16, sh, None)

        plsc.subcore_barrier()

        def start_gather(j, p):
            pltpu.async_copy(h2.at[col_v.at[pl.ds(j * CH, CH)]], buf2.at[p],
                             gsem[p])

        def wait_gather(j, p):
            pltpu.make_async_copy(h2.at[col_v.at[pl.ds(j * CH, CH)]],
                                  buf2.at[p], gsem[p]).wait()

        def start_scatter(j, p):
            pltpu.async_copy(buf2.at[p], acc.at[row_v.at[j]], ssem[p],
                             add=True)

        def wait_scatter(j, p):
            pltpu.make_async_copy(buf2.at[p], acc.at[row_v.at[j]],
                                  ssem[p]).wait()

        def scale(j, p):
            for g in range(CH // 16):
                wg = w_v[pl.ds(j * CH + g * 16, 16)]
                for t in range(16):
                    wv = lax.gather(
                        wg, jnp.full((16, 1), t, jnp.int32),
                        lax.GatherDimensionNumbers(
                            offset_dims=(), collapsed_slice_dims=(0,),
                            start_index_map=(0,)),
                        slice_sizes=(1,),
                        mode=lax.GatherScatterMode.PROMISE_IN_BOUNDS)
                    r = g * 16 + t
                    for q in range(dh // 16):
                        buf2[p, r, pl.ds(q * 16, 16)] = (
                            buf2[p, r, pl.ds(q * 16, 16)] * wv)

        def steady(j, p):
            wait_scatter(j - 1, (p + 2) % 3)
            start_gather(j + 2, (p + 2) % 3)
            wait_gather(j, p)
            scale(j, p)
            start_scatter(j, p)

        start_gather(jnp.int32(0), 0)
        start_gather(jnp.int32(1), 1)
        start_gather(jnp.int32(2), 2)
        wait_gather(jnp.int32(0), 0)
        scale(jnp.int32(0), 0)
        start_scatter(jnp.int32(0), 0)

        ntrip = (nchunk - 3) // 3
        def trip(jj, carry):
            steady(3 * jj + 1, 1)
            steady(3 * jj + 2, 2)
            steady(3 * jj + 3, 0)
            return carry
        lax.fori_loop(0, ntrip, trip, None)
        for j in range(3 * ntrip + 1, nchunk - 2):
            steady(jnp.int32(j), j % 3)
        for j in range(nchunk - 2, nchunk):
            wait_gather(jnp.int32(j), j % 3)
            scale(jnp.int32(j), j % 3)
            start_scatter(jnp.int32(j), j % 3)
        for j in range(nchunk - 3, nchunk):
            wait_scatter(jnp.int32(j), j % 3)

        plsc.subcore_barrier()
        pltpu.sync_copy(acc.at[pl.ds(s * rpt, rpt)], out_h.at[c * NS + s])

    return sc_scatter


def kernel(x, edge_index, edge_weight, W, b):
    n, d = x.shape
    e = edge_weight.shape[0]
    dh = d // NC
    col = edge_index[1].astype(jnp.int32)
    row2 = edge_index[0].astype(jnp.int32).reshape(NS, e // (NS * CH), CH)
    h2 = _linear_split(x, W, b)
    out2 = _build_scatter(n, e, dh)(h2, col, row2, edge_weight)
    out2 = out2.reshape(NC, n, dh)
    return jnp.concatenate([out2[0], out2[1]], axis=1)


# speedup vs baseline: 10.4234x; 1.1960x over previous
import functools

import jax
import jax.numpy as jnp
from jax import lax
from jax.experimental import pallas as pl
from jax.experimental.pallas import tpu as pltpu
from jax.experimental.pallas import tpu_sc as plsc

NC = 2
NS = 16
CH = 80


def _linear_split(x, W, b, blk=1000):
    n, d = x.shape
    dh = d // NC

    def body(x_ref, w_ref, b_ref, o_ref):
        o_ref[...] = lax.dot_general(
            x_ref[...], w_ref[...], (((1,), (1,)), ((), ())),
            preferred_element_type=jnp.float32) + b_ref[0]

    nb = n // blk
    return pl.pallas_call(
        body,
        grid=(nb, NC),
        in_specs=[
            pl.BlockSpec((blk, d), lambda i, p: (i, 0)),
            pl.BlockSpec((dh, d), lambda i, p: (p, 0)),
            pl.BlockSpec((1, 1, dh), lambda i, p: (p, 0, 0)),
        ],
        out_specs=pl.BlockSpec((blk, dh), lambda i, p: (p * nb + i, 0)),
        out_shape=jax.ShapeDtypeStruct((NC * n, dh), jnp.float32),
    )(x, W, b.reshape(NC, 1, dh))


def _build_scatter(n, e, dh):
    ept = e // NS
    nchunk = ept // CH
    rpt = n // NS
    zr = 125
    nz = rpt // zr

    @functools.partial(
        pl.kernel,
        out_type=jax.ShapeDtypeStruct((n, NC * dh), jnp.float32),
        mesh=plsc.VectorSubcoreMesh(core_axis_name="c", subcore_axis_name="s"),
        scratch_types=[
            pltpu.VMEM((ept,), jnp.int32),
            pltpu.VMEM((nchunk, CH), jnp.int32),
            pltpu.VMEM((ept,), jnp.float32),
            pltpu.VMEM((3, CH, dh), jnp.float32),
            pltpu.VMEM((zr, dh), jnp.float32),
            pltpu.VMEM_SHARED((n, dh), jnp.float32),
            pltpu.SemaphoreType.DMA,
            pltpu.SemaphoreType.DMA,
            pltpu.SemaphoreType.DMA,
            pltpu.SemaphoreType.DMA,
            pltpu.SemaphoreType.DMA,
            pltpu.SemaphoreType.DMA,
        ],
        compiler_params=pltpu.CompilerParams(use_tc_tiling_on_sc=False),
    )
    def sc_scatter(h2, col_h, row_h, w_h, out_h, col_v, row_v, w_v, buf2, zb,
                   acc, gsem0, gsem1, gsem2, ssem0, ssem1, ssem2):
        gsem = (gsem0, gsem1, gsem2)
        ssem = (ssem0, ssem1, ssem2)
        c = lax.axis_index("c")
        s = lax.axis_index("s")
        zero16 = jnp.zeros((16,), jnp.float32)

        def zrow(i, carry):
            for q in range(dh // 16):
                zb[i, pl.ds(q * 16, 16)] = zero16
            return carry
        lax.fori_loop(0, zr, zrow, None)
        for k in range(nz):
            pltpu.sync_copy(zb, acc.at[pl.ds(s * rpt + k * zr, zr)])

        base = s * ept
        pltpu.sync_copy(col_h.at[pl.ds(base, ept)], col_v)
        pltpu.sync_copy(row_h.at[s], row_v)
        pltpu.sync_copy(w_h.at[pl.ds(base, ept)], w_v)

        shift = jnp.full((16,), c * n, jnp.int32)

        def sh(i, carry):
            col_v[pl.ds(i * 16, 16)] = col_v[pl.ds(i * 16, 16)] + shift
            return carry
        lax.fori_loop(0, ept // 16, sh, None)

        plsc.subcore_barrier()

        def start_gather(j, p):
            pltpu.async_copy(h2.at[col_v.at[pl.ds(j * CH, CH)]], buf2.at[p],
                             gsem[p])

        def wait_gather(j, p):
            pltpu.make_async_copy(h2.at[col_v.at[pl.ds(j * CH, CH)]],
                                  buf2.at[p], gsem[p]).wait()

        def start_scatter(j, p):
            pltpu.async_copy(buf2.at[p], acc.at[row_v.at[j]], ssem[p],
                             add=True)

        def wait_scatter(j, p):
            pltpu.make_async_copy(buf2.at[p], acc.at[row_v.at[j]],
                                  ssem[p]).wait()

        def scale(j, p):
            for g in range(CH // 16):
                wg = w_v[pl.ds(j * CH + g * 16, 16)]
                for t in range(16):
                    wv = lax.gather(
                        wg, jnp.full((16, 1), t, jnp.int32),
                        lax.GatherDimensionNumbers(
                            offset_dims=(), collapsed_slice_dims=(0,),
                            start_index_map=(0,)),
                        slice_sizes=(1,),
                        mode=lax.GatherScatterMode.PROMISE_IN_BOUNDS)
                    r = g * 16 + t
                    for q in range(dh // 16):
                        buf2[p, r, pl.ds(q * 16, 16)] = (
                            buf2[p, r, pl.ds(q * 16, 16)] * wv)

        def steady(j, p):
            wait_scatter(j - 1, (p + 2) % 3)
            start_gather(j + 2, (p + 2) % 3)
            wait_gather(j, p)
            scale(j, p)
            start_scatter(j, p)

        start_gather(jnp.int32(0), 0)
        start_gather(jnp.int32(1), 1)
        start_gather(jnp.int32(2), 2)
        wait_gather(jnp.int32(0), 0)
        scale(jnp.int32(0), 0)
        start_scatter(jnp.int32(0), 0)

        ntrip = (nchunk - 3) // 3
        def trip(jj, carry):
            steady(3 * jj + 1, 1)
            steady(3 * jj + 2, 2)
            steady(3 * jj + 3, 0)
            return carry
        lax.fori_loop(0, ntrip, trip, None)
        for j in range(3 * ntrip + 1, nchunk - 2):
            steady(jnp.int32(j), j % 3)
        for j in range(nchunk - 2, nchunk):
            wait_gather(jnp.int32(j), j % 3)
            scale(jnp.int32(j), j % 3)
            start_scatter(jnp.int32(j), j % 3)
        for j in range(nchunk - 3, nchunk):
            wait_scatter(jnp.int32(j), j % 3)

        plsc.subcore_barrier()
        pltpu.sync_copy(acc.at[pl.ds(s * rpt, rpt)],
                        out_h.at[pl.ds(s * rpt, rpt), pl.ds(c * dh, dh)])

    return sc_scatter


def kernel(x, edge_index, edge_weight, W, b):
    n, d = x.shape
    e = edge_weight.shape[0]
    dh = d // NC
    col = edge_index[1].astype(jnp.int32)
    row2 = edge_index[0].astype(jnp.int32).reshape(NS, e // (NS * CH), CH)
    h2 = _linear_split(x, W, b)
    return _build_scatter(n, e, dh)(h2, col, row2, edge_weight)
